# initial kernel scaffold (unmeasured)
import jax
import jax.numpy as jnp
from jax import lax
from jax.experimental import pallas as pl
from jax.experimental.pallas import tpu as pltpu

N_DEV = 32
N_ROWS = 512
D_MODEL = 256
N_EXPERTS = 128
E_LOCAL = 4
H = 512
ROWS_PER = N_ROWS // N_DEV


def kernel(x, router_W, route_idx, expert_W, shared_W):
    def body(x_ref, rw_ref, idx_ref, ew_ref, sw_ref, out_ref,
             send_ref, recv_ref, send_sems, recv_sems):
        my_i = lax.axis_index("i")

        barrier_sem = pltpu.get_barrier_semaphore()
        for k in range(1, N_DEV):
            peer = lax.rem(my_i + k, N_DEV)
            pl.semaphore_signal(
                barrier_sem, inc=1,
                device_id=(peer,), device_id_type=pl.DeviceIdType.MESH,
            )
        pl.semaphore_wait(barrier_sem, N_DEV - 1)

        scores = jnp.dot(x_ref[:, :], rw_ref[:, :],
                         preferred_element_type=jnp.float32)
        s_max = jnp.max(scores, axis=-1, keepdims=True)
        p = jnp.exp(scores - s_max)
        probs = p / jnp.sum(p, axis=-1, keepdims=True)
        idx = idx_ref[:, :]
        eids = lax.broadcasted_iota(jnp.int32, (N_ROWS, N_EXPERTS), 1)
        p_sel = jnp.sum(jnp.where(idx == eids, probs, 0.0),
                        axis=-1, keepdims=True)

        x_b = x_ref[:, :].astype(jnp.bfloat16)
        w_all = jnp.concatenate(
            [ew_ref[e, :, :].astype(jnp.bfloat16) for e in range(E_LOCAL)],
            axis=1)
        y_all = jnp.dot(x_b, w_all, preferred_element_type=jnp.float32)
        contrib = jnp.zeros((N_ROWS, H), jnp.float32)
        for e in range(E_LOCAL):
            gate = jnp.where(idx == my_i * E_LOCAL + e, p_sel, 0.0)
            contrib = contrib + gate * y_all[:, e * H:(e + 1) * H]
        send_ref[:, :] = contrib.astype(jnp.bfloat16)

        rdmas = []
        for k in range(1, N_DEV):
            d = lax.rem(my_i + k, N_DEV)
            rdma = pltpu.make_async_remote_copy(
                src_ref=send_ref.at[pl.ds(d * ROWS_PER, ROWS_PER), :],
                dst_ref=recv_ref.at[k],
                send_sem=send_sems.at[k],
                recv_sem=recv_sems.at[k],
                device_id=(d,),
                device_id_type=pl.DeviceIdType.MESH,
            )
            rdma.start()
            rdmas.append(rdma)

        x_mine = x_ref[pl.ds(my_i * ROWS_PER, ROWS_PER), :]
        shared_mine = jnp.dot(x_mine, sw_ref[:, :],
                              preferred_element_type=jnp.float32)
        mine = lax.dynamic_slice(contrib, (my_i * ROWS_PER, 0),
                                 (ROWS_PER, H))

        for rdma in rdmas:
            rdma.wait_recv()
        for rdma in rdmas:
            rdma.wait_send()

        total = jnp.sum(recv_ref[1:, :, :].astype(jnp.float32), axis=0)
        out_ref[:, :] = shared_mine + mine + total

    return pl.pallas_call(
        body,
        out_shape=jax.ShapeDtypeStruct((ROWS_PER, H), jnp.float32),
        in_specs=[pl.BlockSpec(memory_space=pltpu.VMEM)] * 5,
        out_specs=pl.BlockSpec(memory_space=pltpu.VMEM),
        scratch_shapes=[
            pltpu.VMEM((N_ROWS, H), jnp.bfloat16),
            pltpu.VMEM((N_DEV, ROWS_PER, H), jnp.bfloat16),
            pltpu.SemaphoreType.DMA((N_DEV,)),
            pltpu.SemaphoreType.DMA((N_DEV,)),
        ],
        compiler_params=pltpu.CompilerParams(collective_id=0),
    )(x, router_W, route_idx, expert_W, shared_W)


# baseline (device time: 23017 ns/iter reference)
import jax
import jax.numpy as jnp
from jax import lax
from jax.experimental import pallas as pl
from jax.experimental.pallas import tpu as pltpu

N_DEV = 32
N_ROWS = 512
D_MODEL = 256
N_EXPERTS = 128
E_LOCAL = 4
H = 512
ROWS_PER = N_ROWS // N_DEV


def kernel(x, router_W, route_idx, expert_W, shared_W):
    def body(x_ref, rw_ref, idx_ref, ew_ref, sw_ref, out_ref,
             send_ref, recv_ref, send_sems, recv_sems):
        my_i = lax.axis_index("i")

        barrier_sem = pltpu.get_barrier_semaphore()
        for k in range(1, N_DEV):
            peer = lax.rem(my_i + k, N_DEV)
            pl.semaphore_signal(
                barrier_sem, inc=1,
                device_id=(peer,), device_id_type=pl.DeviceIdType.MESH,
            )
        pl.semaphore_wait(barrier_sem, N_DEV - 1)

        scores = jnp.dot(x_ref[:, :], rw_ref[:, :],
                         preferred_element_type=jnp.float32)
        s_max = jnp.max(scores, axis=-1, keepdims=True)
        p = jnp.exp(scores - s_max)
        probs = p / jnp.sum(p, axis=-1, keepdims=True)
        idx = idx_ref[:, :]
        eids = lax.broadcasted_iota(jnp.int32, (N_ROWS, N_EXPERTS), 1)
        p_sel = jnp.sum(jnp.where(idx == eids, probs, 0.0),
                        axis=-1, keepdims=True)

        x_b = x_ref[:, :].astype(jnp.bfloat16)
        w_all = jnp.concatenate(
            [ew_ref[e, :, :].astype(jnp.bfloat16) for e in range(E_LOCAL)],
            axis=1)
        y_all = jnp.dot(x_b, w_all, preferred_element_type=jnp.float32)
        contrib = jnp.zeros((N_ROWS, H), jnp.float32)
        for e in range(E_LOCAL):
            gate = jnp.where(idx == my_i * E_LOCAL + e, p_sel, 0.0)
            contrib = contrib + gate * y_all[:, e * H:(e + 1) * H]
        send_ref[:, :] = contrib.astype(jnp.bfloat16)

        rdmas = []
        for k in range(1, N_DEV):
            d = lax.rem(my_i + k, N_DEV)
            rdma = pltpu.make_async_remote_copy(
                src_ref=send_ref.at[pl.ds(d * ROWS_PER, ROWS_PER), :],
                dst_ref=recv_ref.at[k],
                send_sem=send_sems.at[k],
                recv_sem=recv_sems.at[k],
                device_id=(d,),
                device_id_type=pl.DeviceIdType.MESH,
            )
            rdma.start()
            rdmas.append(rdma)

        x_mine = x_ref[pl.ds(my_i * ROWS_PER, ROWS_PER), :]
        shared_mine = jnp.dot(x_mine, sw_ref[:, :],
                              preferred_element_type=jnp.float32)
        mine = send_ref[pl.ds(my_i * ROWS_PER, ROWS_PER), :].astype(
            jnp.float32)

        for rdma in rdmas:
            rdma.wait_recv()
        for rdma in rdmas:
            rdma.wait_send()

        total = jnp.sum(recv_ref[1:, :, :].astype(jnp.float32), axis=0)
        out_ref[:, :] = shared_mine + mine + total

    return pl.pallas_call(
        body,
        out_shape=jax.ShapeDtypeStruct((ROWS_PER, H), jnp.float32),
        in_specs=[pl.BlockSpec(memory_space=pltpu.VMEM)] * 5,
        out_specs=pl.BlockSpec(memory_space=pltpu.VMEM),
        scratch_shapes=[
            pltpu.VMEM((N_ROWS, H), jnp.bfloat16),
            pltpu.VMEM((N_DEV, ROWS_PER, H), jnp.bfloat16),
            pltpu.SemaphoreType.DMA((N_DEV,)),
            pltpu.SemaphoreType.DMA((N_DEV,)),
        ],
        compiler_params=pltpu.CompilerParams(collective_id=0),
    )(x, router_W, route_idx, expert_W, shared_W)


# device time: 22934 ns/iter; 1.0036x vs baseline; 1.0036x over previous
import jax
import jax.numpy as jnp
from jax import lax
from jax.experimental import pallas as pl
from jax.experimental.pallas import tpu as pltpu

N_DEV = 32
N_ROWS = 512
D_MODEL = 256
N_EXPERTS = 128
E_LOCAL = 4
H = 512
ROWS_PER = N_ROWS // N_DEV


def kernel(x, router_W, route_idx, expert_W, shared_W):
    def body(x_ref, rw_ref, idx_ref, ew_ref, sw_ref, out_ref,
             send_ref, recv_ref, send_sems, recv_sems):
        my_i = lax.axis_index("i")

        barrier_sem = pltpu.get_barrier_semaphore()
        for k in range(1, N_DEV):
            peer = lax.rem(my_i + k, N_DEV)
            pl.semaphore_signal(
                barrier_sem, inc=1,
                device_id=(peer,), device_id_type=pl.DeviceIdType.MESH,
            )
        pl.semaphore_wait(barrier_sem, N_DEV - 1)

        scores = jnp.dot(x_ref[:, :], rw_ref[:, :],
                         preferred_element_type=jnp.float32)
        s_max = jnp.max(scores, axis=-1, keepdims=True)
        p = jnp.exp(scores - s_max)
        probs = p / jnp.sum(p, axis=-1, keepdims=True)
        idx = idx_ref[:, :]
        eids = lax.broadcasted_iota(jnp.int32, (N_ROWS, N_EXPERTS), 1)
        p_sel = jnp.sum(jnp.where(idx == eids, probs, 0.0),
                        axis=-1, keepdims=True)

        x_b = x_ref[:, :].astype(jnp.bfloat16)
        contrib = jnp.zeros((N_ROWS, H), jnp.float32)
        for e in range(E_LOCAL):
            y = jnp.dot(x_b, ew_ref[e, :, :].astype(jnp.bfloat16),
                        preferred_element_type=jnp.float32)
            gate = jnp.where(idx == my_i * E_LOCAL + e, p_sel, 0.0)
            contrib = contrib + gate * y
        send_ref[:, :] = contrib.astype(jnp.bfloat16)

        rdmas = []
        for k in range(1, N_DEV):
            d = lax.rem(my_i + k, N_DEV)
            rdma = pltpu.make_async_remote_copy(
                src_ref=send_ref.at[pl.ds(d * ROWS_PER, ROWS_PER), :],
                dst_ref=recv_ref.at[k],
                send_sem=send_sems.at[k],
                recv_sem=recv_sems.at[k],
                device_id=(d,),
                device_id_type=pl.DeviceIdType.MESH,
            )
            rdma.start()
            rdmas.append(rdma)

        x_mine = x_ref[pl.ds(my_i * ROWS_PER, ROWS_PER), :]
        shared_mine = jnp.dot(x_mine, sw_ref[:, :],
                              preferred_element_type=jnp.float32)
        mine = send_ref[pl.ds(my_i * ROWS_PER, ROWS_PER), :].astype(
            jnp.float32)

        for rdma in rdmas:
            rdma.wait_recv()
        for rdma in rdmas:
            rdma.wait_send()

        total = jnp.sum(recv_ref[1:, :, :].astype(jnp.float32), axis=0)
        out_ref[:, :] = shared_mine + mine + total

    return pl.pallas_call(
        body,
        out_shape=jax.ShapeDtypeStruct((ROWS_PER, H), jnp.float32),
        in_specs=[pl.BlockSpec(memory_space=pltpu.VMEM)] * 5,
        out_specs=pl.BlockSpec(memory_space=pltpu.VMEM),
        scratch_shapes=[
            pltpu.VMEM((N_ROWS, H), jnp.bfloat16),
            pltpu.VMEM((N_DEV, ROWS_PER, H), jnp.bfloat16),
            pltpu.SemaphoreType.DMA((N_DEV,)),
            pltpu.SemaphoreType.DMA((N_DEV,)),
        ],
        compiler_params=pltpu.CompilerParams(collective_id=0),
    )(x, router_W, route_idx, expert_W, shared_W)
